# baseline (speedup 1.0000x reference)
import jax
import jax.numpy as jnp
from jax import lax
from jax.experimental import pallas as pl
from jax.experimental.pallas import tpu as pltpu
from jax.experimental.pallas import tpu_sc as plsc

_NUM_USER = 30000
_NUM_ITEM = 20000
_N = _NUM_USER + _NUM_ITEM
_NP = 51200
_H = 32
_E = 800000
_C = 128
_BLK = 1024
_NCPB = _BLK // _C
_NBLK = 50
_PER_TILE = _NBLK * _BLK
_E_PAD = 16 * _PER_TILE
_RPT = _NP // 16
_RC = 128
_NRC = _RPT // _RC


def _sc_body(row_ref, col_ref, val_ref, x0_ref, out_ref, xt,
             acc, g0, g1, g2, g3, zbuf,
             colb0, rowb0, valb0, colb1, rowb1, valb1,
             sg0, sg1, sg2, sg3, ss0, ss1, ss2, ss3, si0, si1):
    c = lax.axis_index("c")
    s = lax.axis_index("s")
    r0 = s * _RPT
    blk0 = s * _NBLK
    xoff = c * jnp.int32(_NP)
    zero16 = jnp.zeros((16,), jnp.float32)
    lane = jnp.arange(16, dtype=jnp.int32)

    gbufs = (g0, g1, g2, g3)
    gsems = (sg0, sg1, sg2, sg3)
    ssems = (ss0, ss1, ss2, ss3)
    idxbufs = ((colb0, rowb0, valb0, si0), (colb1, rowb1, valb1, si1))

    @pl.loop(0, 32)
    def _(i):
        zbuf[i, pl.ds(0, 16)] = zero16
        zbuf[i, pl.ds(16, 16)] = zero16

    def _zero_acc(r):
        for j in range(4):
            pltpu.sync_copy(zbuf, acc.at[pl.ds(r + 32 * j, 32)])

    def _fire_idx(b, buf):
        cb, rb, vb, sem = buf
        base = (blk0 + b) * _BLK
        pltpu.make_async_copy(col_ref.at[pl.ds(base, _BLK)], cb, sem).start()
        pltpu.make_async_copy(val_ref.at[pl.ds(base, _BLK)], vb, sem).start()
        pltpu.make_async_copy(
            row_ref.at[pl.ds((blk0 + b) * _NCPB, _NCPB)], rb, sem).start()

    def _wait_idx(buf):
        cb, rb, vb, sem = buf
        pltpu.make_async_copy(col_ref.at[pl.ds(0, _BLK)], cb, sem).wait()
        pltpu.make_async_copy(val_ref.at[pl.ds(0, _BLK)], vb, sem).wait()
        pltpu.make_async_copy(row_ref.at[pl.ds(0, _NCPB)], rb, sem).wait()

    def _scale(gb, vb, v0):
        @pl.loop(0, _C // 16)
        def _(g):
            vals = vb[pl.ds(v0 + g * 16, 16)]
            for t in range(16):
                e = g * 16 + t
                gb[e, pl.ds(0, 16)] = gb[e, pl.ds(0, 16)] * vals[t]
                gb[e, pl.ds(16, 16)] = gb[e, pl.ds(16, 16)] * vals[t]

    def _edges():
        xoff16 = jnp.full((16,), xoff, jnp.int32)

        _fire_idx(0, idxbufs[0])

        def _block(b, buf, nxt):
            cb, rb, vb, _ = buf
            _wait_idx(buf)

            @pl.when(b + 1 < _NBLK)
            def _():
                _fire_idx(b + 1, nxt)

            @pl.loop(0, _BLK // 16)
            def _(m):
                cb[pl.ds(m * 16, 16)] = cb[pl.ds(m * 16, 16)] + xoff16

            gcp = {}
            scp = {}

            def fire_gather(k):
                sl = k % 4
                cp = pltpu.make_async_copy(
                    xt.at[cb.at[pl.ds(k * _C, _C)]], gbufs[sl], gsems[sl])
                cp.start()
                gcp[k] = cp

            def fire_scatter(k):
                sl = k % 4
                cp = pltpu.make_async_copy(
                    gbufs[sl], acc.at[rb.at[k]], ssems[sl])
                cp.start(add=True)
                scp[k] = cp

            fire_gather(0)
            fire_gather(1)
            fire_gather(2)
            for k in range(_NCPB):
                gcp[k].wait()
                _scale(gbufs[k % 4], vb, k * _C)
                fire_scatter(k)
                if k + 3 < _NCPB:
                    if k >= 1:
                        scp[k - 1].wait()
                    fire_gather(k + 3)
            for k in range(_NCPB - 4, _NCPB):
                scp[k].wait()

        @pl.loop(0, _NBLK, step=2)
        def _(b):
            _block(b, idxbufs[0], idxbufs[1])
            _block(b + 1, idxbufs[1], idxbufs[0])

        plsc.subcore_barrier()

    @pl.loop(0, _NRC)
    def _(k):
        r = r0 + k * _RC

        @pl.loop(0, _RC // 16)
        def _(m):
            node = lane + (r + m * 16)
            rowb0[0, pl.ds(m * 16, 16)] = jnp.minimum(node, _N - 1) * 2 + c

        pltpu.sync_copy(x0_ref.at[rowb0.at[0]], g0)
        pltpu.sync_copy(g0, xt.at[pl.ds(xoff + r, _RC)])
        pltpu.sync_copy(g0, out_ref.at[pl.ds(xoff + r, _RC)])
        _zero_acc(r)

    plsc.subcore_barrier()

    @pl.loop(0, 3)
    def _(l):
        _edges()

        lvec = jnp.full((16,), l, jnp.int32)
        sc16 = jnp.where(lvec == 2, jnp.float32(0.25), jnp.float32(1.0))

        @pl.loop(0, _NRC)
        def _(k):
            r = r0 + k * _RC
            pltpu.sync_copy(acc.at[pl.ds(r, _RC)], g0)
            pltpu.sync_copy(g0, xt.at[pl.ds(xoff + r, _RC)])
            _zero_acc(r)
            pltpu.sync_copy(out_ref.at[pl.ds(xoff + r, _RC)], g1)

            @pl.loop(0, _RC, unroll=8)
            def _(i):
                a0 = (g1[i, pl.ds(0, 16)] + g0[i, pl.ds(0, 16)]) * sc16
                a1 = (g1[i, pl.ds(16, 16)] + g0[i, pl.ds(16, 16)]) * sc16
                g1[i, pl.ds(0, 16)] = a0
                g1[i, pl.ds(16, 16)] = a1

            pltpu.sync_copy(g1, out_ref.at[pl.ds(xoff + r, _RC)])

        plsc.subcore_barrier()


@jax.jit
def _sc_call(row_p, col_p, val_p, x0il):
    mesh = plsc.VectorSubcoreMesh(core_axis_name="c", subcore_axis_name="s")
    f = pl.kernel(
        _sc_body,
        out_type=jax.ShapeDtypeStruct((2 * _NP, _H), jnp.float32),
        mesh=mesh,
        compiler_params=pltpu.CompilerParams(use_tc_tiling_on_sc=False),
        scratch_types=[
            pltpu.HBM((2 * _NP, _H), jnp.float32),
            pltpu.VMEM_SHARED((_NP, _H), jnp.float32),
            pltpu.VMEM((_C, _H), jnp.float32),
            pltpu.VMEM((_C, _H), jnp.float32),
            pltpu.VMEM((_C, _H), jnp.float32),
            pltpu.VMEM((_C, _H), jnp.float32),
            pltpu.VMEM((32, _H), jnp.float32),
            pltpu.VMEM((_BLK,), jnp.int32),
            pltpu.VMEM((_NCPB, _C), jnp.int32),
            pltpu.VMEM((_BLK,), jnp.float32),
            pltpu.VMEM((_BLK,), jnp.int32),
            pltpu.VMEM((_NCPB, _C), jnp.int32),
            pltpu.VMEM((_BLK,), jnp.float32),
            pltpu.SemaphoreType.DMA,
            pltpu.SemaphoreType.DMA,
            pltpu.SemaphoreType.DMA,
            pltpu.SemaphoreType.DMA,
            pltpu.SemaphoreType.DMA,
            pltpu.SemaphoreType.DMA,
            pltpu.SemaphoreType.DMA,
            pltpu.SemaphoreType.DMA,
            pltpu.SemaphoreType.DMA,
            pltpu.SemaphoreType.DMA,
        ],
    )
    return f(row_p, col_p, val_p, x0il)


def kernel(adj_indices, adj_values, user_emb_w, item_emb_w):
    x0 = jnp.concatenate([user_emb_w, item_emb_w], axis=0)
    x0il = x0.reshape(2 * _N, _H)
    pad = _E_PAD - _E
    pidx = jnp.arange(pad, dtype=jnp.int32) % _N
    row_p = jnp.concatenate([adj_indices[0].astype(jnp.int32), pidx])
    col_p = jnp.concatenate([adj_indices[1].astype(jnp.int32), pidx])
    val_p = jnp.concatenate([adj_values, jnp.zeros((pad,), jnp.float32)])
    col_s, row_s, val_s = lax.sort((col_p, row_p, val_p), num_keys=1)
    row2d = row_s.reshape(_E_PAD // _C, _C)
    outil = _sc_call(row2d, col_s, val_s, x0il)
    halves = outil.reshape(2, _NP, _H)
    mean = jnp.concatenate([halves[0, :_N], halves[1, :_N]], axis=1)
    return mean[:_NUM_USER], mean[_NUM_USER:]


# speedup vs baseline: 2.6539x; 2.6539x over previous
import jax
import jax.numpy as jnp
from jax import lax
from jax.experimental import pallas as pl
from jax.experimental.pallas import tpu as pltpu
from jax.experimental.pallas import tpu_sc as plsc

_NUM_USER = 30000
_NUM_ITEM = 20000
_N = _NUM_USER + _NUM_ITEM
_NP = 51200
_H = 32
_E = 800000
_C = 128
_BLK = 1024
_NCPB = _BLK // _C
_NBLK = 50
_PER_TILE = _NBLK * _BLK
_E_PAD = 16 * _PER_TILE
_RPT = _NP // 16
_RC = 128
_NRC = _RPT // _RC


def _sc_body(row_ref, col_ref, val_ref, x0_ref, out_ref, xt,
             acc, g0, g1, g2, g3, zbuf,
             colb0, rowb0, valb0, colb1, rowb1, valb1,
             sg0, sg1, sg2, sg3, ss0, ss1, ss2, ss3, si0, si1):
    c = lax.axis_index("c")
    s = lax.axis_index("s")
    r0 = s * _RPT
    blk0 = s * _NBLK
    xoff = c * jnp.int32(_NP)
    zero16 = jnp.zeros((16,), jnp.float32)
    lane = jnp.arange(16, dtype=jnp.int32)

    gbufs = (g0, g1, g2, g3)
    gsems = (sg0, sg1, sg2, sg3)
    ssems = (ss0, ss1, ss2, ss3)
    idxbufs = ((colb0, rowb0, valb0, si0), (colb1, rowb1, valb1, si1))

    @pl.loop(0, 32)
    def _(i):
        zbuf[i, pl.ds(0, 16)] = zero16
        zbuf[i, pl.ds(16, 16)] = zero16

    def _zero_acc(r):
        for j in range(4):
            pltpu.sync_copy(zbuf, acc.at[pl.ds(r + 32 * j, 32)])

    def _fire_idx(b, buf):
        cb, rb, vb, sem = buf
        base = (blk0 + b) * _BLK
        pltpu.make_async_copy(col_ref.at[pl.ds(base, _BLK)], cb, sem).start()
        pltpu.make_async_copy(val_ref.at[pl.ds(base, _BLK)], vb, sem).start()
        pltpu.make_async_copy(
            row_ref.at[pl.ds((blk0 + b) * _NCPB, _NCPB)], rb, sem).start()

    def _wait_idx(buf):
        cb, rb, vb, sem = buf
        pltpu.make_async_copy(col_ref.at[pl.ds(0, _BLK)], cb, sem).wait()
        pltpu.make_async_copy(val_ref.at[pl.ds(0, _BLK)], vb, sem).wait()
        pltpu.make_async_copy(row_ref.at[pl.ds(0, _NCPB)], rb, sem).wait()

    def _scale(gb, vb, v0):
        @pl.loop(0, _C // 16)
        def _(g):
            vals = vb[pl.ds(v0 + g * 16, 16)]
            for t in range(16):
                e = g * 16 + t
                gb[e, pl.ds(0, 16)] = gb[e, pl.ds(0, 16)] * vals[t]
                gb[e, pl.ds(16, 16)] = gb[e, pl.ds(16, 16)] * vals[t]

    def _edges():
        xoff16 = jnp.full((16,), xoff, jnp.int32)

        _fire_idx(0, idxbufs[0])

        def _block(b, buf, nxt):
            cb, rb, vb, _ = buf
            _wait_idx(buf)

            @pl.when(b + 1 < _NBLK)
            def _():
                _fire_idx(b + 1, nxt)

            @pl.loop(0, _BLK // 16)
            def _(m):
                cb[pl.ds(m * 16, 16)] = cb[pl.ds(m * 16, 16)] + xoff16

            gcp = {}
            scp = {}

            def fire_gather(k):
                sl = k % 4
                cp = pltpu.make_async_copy(
                    xt.at[cb.at[pl.ds(k * _C, _C)]], gbufs[sl], gsems[sl])
                cp.start()
                gcp[k] = cp

            def fire_scatter(k):
                sl = k % 4
                cp = pltpu.make_async_copy(
                    gbufs[sl], acc.at[rb.at[k]], ssems[sl])
                cp.start(add=True)
                scp[k] = cp

            fire_gather(0)
            fire_gather(1)
            fire_gather(2)
            for k in range(_NCPB):
                gcp[k].wait()
                _scale(gbufs[k % 4], vb, k * _C)
                fire_scatter(k)
                if k + 3 < _NCPB:
                    if k >= 1:
                        scp[k - 1].wait()
                    fire_gather(k + 3)
            for k in range(_NCPB - 4, _NCPB):
                scp[k].wait()

        @pl.loop(0, _NBLK, step=2)
        def _(b):
            _block(b, idxbufs[0], idxbufs[1])
            _block(b + 1, idxbufs[1], idxbufs[0])

        plsc.subcore_barrier()

    @pl.loop(0, _NRC)
    def _(k):
        r = r0 + k * _RC

        @pl.loop(0, _RC // 16)
        def _(m):
            node = lane + (r + m * 16)
            rowb0[0, pl.ds(m * 16, 16)] = jnp.minimum(node, _N - 1) * 2 + c

        pltpu.sync_copy(x0_ref.at[rowb0.at[0]], g0)
        pltpu.sync_copy(g0, xt.at[pl.ds(xoff + r, _RC)])
        pltpu.sync_copy(g0, out_ref.at[pl.ds(xoff + r, _RC)])
        _zero_acc(r)

    plsc.subcore_barrier()

    @pl.loop(0, 3)
    def _(l):
        _edges()

        lvec = jnp.full((16,), l, jnp.int32)
        sc16 = jnp.where(lvec == 2, jnp.float32(0.25), jnp.float32(1.0))

        @pl.loop(0, _NRC)
        def _(k):
            r = r0 + k * _RC
            pltpu.sync_copy(acc.at[pl.ds(r, _RC)], g0)
            pltpu.sync_copy(g0, xt.at[pl.ds(xoff + r, _RC)])
            _zero_acc(r)
            pltpu.sync_copy(out_ref.at[pl.ds(xoff + r, _RC)], g1)

            @pl.loop(0, _RC, unroll=8)
            def _(i):
                a0 = (g1[i, pl.ds(0, 16)] + g0[i, pl.ds(0, 16)]) * sc16
                a1 = (g1[i, pl.ds(16, 16)] + g0[i, pl.ds(16, 16)]) * sc16
                g1[i, pl.ds(0, 16)] = a0
                g1[i, pl.ds(16, 16)] = a1

            pltpu.sync_copy(g1, out_ref.at[pl.ds(xoff + r, _RC)])

        plsc.subcore_barrier()


@jax.jit
def _sc_call(row_p, col_p, val_p, x0il):
    mesh = plsc.VectorSubcoreMesh(core_axis_name="c", subcore_axis_name="s")
    f = pl.kernel(
        _sc_body,
        out_type=jax.ShapeDtypeStruct((2 * _NP, _H), jnp.float32),
        mesh=mesh,
        compiler_params=pltpu.CompilerParams(use_tc_tiling_on_sc=False),
        scratch_types=[
            pltpu.HBM((2 * _NP, _H), jnp.float32),
            pltpu.VMEM_SHARED((_NP, _H), jnp.float32),
            pltpu.VMEM((_C, _H), jnp.float32),
            pltpu.VMEM((_C, _H), jnp.float32),
            pltpu.VMEM((_C, _H), jnp.float32),
            pltpu.VMEM((_C, _H), jnp.float32),
            pltpu.VMEM((32, _H), jnp.float32),
            pltpu.VMEM((_BLK,), jnp.int32),
            pltpu.VMEM((_NCPB, _C), jnp.int32),
            pltpu.VMEM((_BLK,), jnp.float32),
            pltpu.VMEM((_BLK,), jnp.int32),
            pltpu.VMEM((_NCPB, _C), jnp.int32),
            pltpu.VMEM((_BLK,), jnp.float32),
            pltpu.SemaphoreType.DMA,
            pltpu.SemaphoreType.DMA,
            pltpu.SemaphoreType.DMA,
            pltpu.SemaphoreType.DMA,
            pltpu.SemaphoreType.DMA,
            pltpu.SemaphoreType.DMA,
            pltpu.SemaphoreType.DMA,
            pltpu.SemaphoreType.DMA,
            pltpu.SemaphoreType.DMA,
            pltpu.SemaphoreType.DMA,
        ],
    )
    return f(row_p, col_p, val_p, x0il)


def kernel(adj_indices, adj_values, user_emb_w, item_emb_w):
    x0 = jnp.concatenate([user_emb_w, item_emb_w], axis=0)
    x0il = x0.reshape(2 * _N, _H)
    pad = _E_PAD - _E
    pidx = jnp.arange(pad, dtype=jnp.int32) % _N
    row_p = jnp.concatenate([adj_indices[0].astype(jnp.int32), pidx])
    col_p = jnp.concatenate([adj_indices[1].astype(jnp.int32), pidx])
    val_p = jnp.concatenate([adj_values, jnp.zeros((pad,), jnp.float32)])
    row2d = row_p.reshape(_E_PAD // _C, _C)
    outil = _sc_call(row2d, col_p, val_p, x0il)
    halves = outil.reshape(2, _NP, _H)
    mean = jnp.concatenate([halves[0, :_N], halves[1, :_N]], axis=1)
    return mean[:_NUM_USER], mean[_NUM_USER:]
